# baseline (speedup 1.0000x reference)
import functools

import jax
import jax.numpy as jnp
from jax import lax
from jax.experimental import pallas as pl
from jax.experimental.pallas import tpu as pltpu
from jax.experimental.pallas import tpu_sc as plsc

N = 10000
E = 320000
D = 128
C = 47
NPAD = 10240
NC, NS = 2, 16
NW = NC * NS
CHUNK = 112
CPW = 96
EPAD = NW * CHUNK * CPW
GRP = 8
NGRP = CPW // GRP
NBUF = 3
RPS = NPAD // NS
BLK = 512


def _mesh():
    return plsc.VectorSubcoreMesh(core_axis_name="c", subcore_axis_name="s")


def _sc_segment_sum(table, src3, dst3, zrows):
    @functools.partial(
        pl.kernel,
        out_type=jax.ShapeDtypeStruct((NC, NPAD, D), jnp.float32),
        mesh=_mesh(),
        scratch_types=[
            pltpu.VMEM((2, GRP, CHUNK), jnp.int32),
            pltpu.VMEM((2, GRP, CHUNK), jnp.int32),
            pltpu.VMEM((NBUF, CHUNK, D), jnp.float32),
            pltpu.VMEM_SHARED((NPAD, D), jnp.float32),
            [pltpu.SemaphoreType.DMA] * NBUF,
            [pltpu.SemaphoreType.DMA] * 4,
        ],
    )
    def k(table_hbm, src_hbm, dst_hbm, z_hbm, out_hbm, sgrp, dgrp, rows, acc,
          gsems, isems):
        c = lax.axis_index("c")
        s = lax.axis_index("s")
        wid = s * NC + c

        def sload(g, slot):
            return pltpu.make_async_copy(
                src_hbm.at[wid].at[pl.ds(g * GRP, GRP)], sgrp.at[slot],
                isems[slot])

        def dload(g, slot):
            return pltpu.make_async_copy(
                dst_hbm.at[wid].at[pl.ds(g * GRP, GRP)], dgrp.at[slot],
                isems[2 + slot])

        pltpu.sync_copy(z_hbm, acc.at[pl.ds(s * RPS, RPS)])
        sload(0, 0).start()
        dload(0, 0).start()
        plsc.subcore_barrier()

        @pl.loop(0, NGRP // 2)
        def _(q):
            for par in range(2):
                g = 2 * q + par
                sload(g, par).wait()
                dload(g, par).wait()

                @pl.when(g + 1 < NGRP)
                def _():
                    sload(g + 1, 1 - par).start()
                    dload(g + 1, 1 - par).start()

                handles = [
                    pltpu.async_copy(table_hbm.at[sgrp.at[par].at[b]],
                                     rows.at[b], gsems[b])
                    for b in range(NBUF)
                ]
                for k_ in range(GRP):
                    b = k_ % NBUF
                    handles[b].wait()
                    pltpu.sync_copy(rows.at[b], acc.at[dgrp.at[par].at[k_]],
                                    add=True)
                    if k_ + NBUF < GRP:
                        handles[b] = pltpu.async_copy(
                            table_hbm.at[sgrp.at[par].at[k_ + NBUF]],
                            rows.at[b], gsems[b])

        plsc.subcore_barrier()
        pltpu.sync_copy(acc.at[pl.ds(s * RPS, RPS)],
                        out_hbm.at[c].at[pl.ds(s * RPS, RPS)])

    return k(table, src3, dst3, zrows)


def _sc_degree(dst3, ones_rows, zrows):
    @functools.partial(
        pl.kernel,
        out_type=jax.ShapeDtypeStruct((NC, NPAD, D), jnp.float32),
        mesh=_mesh(),
        scratch_types=[
            pltpu.VMEM((CPW, CHUNK), jnp.int32),
            pltpu.VMEM((CHUNK, D), jnp.float32),
            pltpu.VMEM_SHARED((NPAD, D), jnp.float32),
        ],
    )
    def k(dst_hbm, ones_hbm, z_hbm, out_hbm, didx, ones_v, acc):
        c = lax.axis_index("c")
        s = lax.axis_index("s")
        wid = s * NC + c

        pltpu.sync_copy(z_hbm, acc.at[pl.ds(s * RPS, RPS)])
        pltpu.sync_copy(dst_hbm.at[wid], didx)
        pltpu.sync_copy(ones_hbm, ones_v)
        plsc.subcore_barrier()

        @pl.loop(0, CPW)
        def _(j):
            pltpu.sync_copy(ones_v, acc.at[didx.at[j]], add=True)

        plsc.subcore_barrier()
        pltpu.sync_copy(acc.at[pl.ds(s * RPS, RPS)],
                        out_hbm.at[c].at[pl.ds(s * RPS, RPS)])

    return k(dst3, ones_rows, zrows)


def _dot(a, b):
    return jnp.dot(a, b, preferred_element_type=jnp.float32)


def _layer1_body(x_ref, p_ref, dp_ref, ws_ref, wn_ref, b_ref, o_ref):
    agg = p_ref[0] + p_ref[1]
    deg = dp_ref[0][:, :1] + dp_ref[1][:, :1]
    recip = 1.0 / jnp.maximum(deg, 1.0)
    mean = agg * recip
    o_ref[...] = jnp.maximum(_dot(x_ref[...], ws_ref[...])
                             + _dot(mean, wn_ref[...]) + b_ref[...], 0.0)


def _layer2_body(h_ref, p_ref, dp_ref, ws_ref, wn_ref, b_ref, mwin_ref,
                 mbin_ref, r1w1_ref, r1b1_ref, r1w2_ref, r1b2_ref, r2w1_ref,
                 r2b1_ref, r2w2_ref, r2b2_ref, wout_ref, bout_ref, o_ref):
    agg = p_ref[0] + p_ref[1]
    deg = dp_ref[0][:, :1] + dp_ref[1][:, :1]
    recip = 1.0 / jnp.maximum(deg, 1.0)
    mean = agg * recip
    z = _dot(h_ref[...], ws_ref[...]) + _dot(mean, wn_ref[...]) + b_ref[...]
    t = jnp.maximum(_dot(z, mwin_ref[...]) + mbin_ref[...], 0.0)
    t = t + _dot(jnp.maximum(_dot(t, r1w1_ref[...]) + r1b1_ref[...], 0.0),
                 r1w2_ref[...]) + r1b2_ref[...]
    t = t + _dot(jnp.maximum(_dot(t, r2w1_ref[...]) + r2b1_ref[...], 0.0),
                 r2w2_ref[...]) + r2b2_ref[...]
    o_ref[...] = _dot(t, wout_ref[...]) + bout_ref[...]


def _row_spec(width):
    return pl.BlockSpec((BLK, width), lambda i: (i, 0))


def _part_spec():
    return pl.BlockSpec((NC, BLK, D), lambda i: (0, i, 0))


def _w_spec(shape):
    return pl.BlockSpec(shape, lambda i: (0,) * len(shape))


def _tc_layer1(xp, parts, degparts, ws, wn, b):
    return pl.pallas_call(
        _layer1_body,
        grid=(NPAD // BLK,),
        in_specs=[_row_spec(D), _part_spec(), _part_spec(), _w_spec((D, D)),
                  _w_spec((D, D)), _w_spec((1, D))],
        out_specs=_row_spec(D),
        out_shape=jax.ShapeDtypeStruct((NPAD, D), jnp.float32),
    )(xp, parts, degparts, ws, wn, b)


def _tc_layer2(h, parts, degparts, ws, wn, b, mwin, mbin, r1w1, r1b1, r1w2,
               r1b2, r2w1, r2b1, r2w2, r2b2, wout, bout):
    wspecs = [_w_spec((D, D)), _w_spec((D, D)), _w_spec((1, D)),
              _w_spec((D, D)), _w_spec((1, D)),
              _w_spec((D, D)), _w_spec((1, D)), _w_spec((D, D)),
              _w_spec((1, D)),
              _w_spec((D, D)), _w_spec((1, D)), _w_spec((D, D)),
              _w_spec((1, D)),
              _w_spec((D, D)), _w_spec((1, D))]
    return pl.pallas_call(
        _layer2_body,
        grid=(NPAD // BLK,),
        in_specs=[_row_spec(D), _part_spec(), _part_spec()] + wspecs,
        out_specs=_row_spec(D),
        out_shape=jax.ShapeDtypeStruct((NPAD, D), jnp.float32),
    )(h, parts, degparts, ws, wn, b, mwin, mbin, r1w1, r1b1, r1w2, r1b2,
      r2w1, r2b1, r2w2, r2b2, wout, bout)


def kernel(x, edge_index, s1_wself, s1_wneigh, s1_b, s2_wself, s2_wneigh,
           s2_b, m_win, m_bin, r1_w1, r1_b1, r1_w2, r1_b2, r2_w1, r2_b1,
           r2_w2, r2_b2, m_wout, m_bout):
    src = edge_index[0]
    dst = edge_index[1]
    pad_e = EPAD - E
    pad_src = jnp.arange(pad_e, dtype=jnp.int32) % N
    src3 = jnp.concatenate([src, pad_src]).reshape(NW, CPW, CHUNK)
    pad_dst = N + jnp.arange(pad_e, dtype=jnp.int32) % (NPAD - N)
    dst3 = jnp.concatenate([dst, pad_dst]).reshape(NW, CPW, CHUNK)

    xp = jnp.zeros((NPAD, D), jnp.float32).at[:N].set(x)
    zrows = jnp.zeros((RPS, D), jnp.float32)
    ones_rows = jnp.ones((CHUNK, D), jnp.float32)

    b1 = s1_b.reshape(1, D)
    b2 = s2_b.reshape(1, D)
    mbin = m_bin.reshape(1, D)
    r1b1 = r1_b1.reshape(1, D)
    r1b2 = r1_b2.reshape(1, D)
    r2b1 = r2_b1.reshape(1, D)
    r2b2 = r2_b2.reshape(1, D)
    wout = jnp.zeros((D, D), jnp.float32).at[:, :C].set(m_wout)
    bout = jnp.zeros((1, D), jnp.float32).at[0, :C].set(m_bout)

    degparts = _sc_degree(dst3, ones_rows, zrows)
    parts1 = _sc_segment_sum(xp, src3, dst3, zrows)
    h = _tc_layer1(xp, parts1, degparts, s1_wself, s1_wneigh, b1)
    parts2 = _sc_segment_sum(h, src3, dst3, zrows)
    out = _tc_layer2(h, parts2, degparts, s2_wself, s2_wneigh, b2, m_win,
                     mbin, r1_w1, r1b1, r1_w2, r1b2, r2_w1, r2b1, r2_w2,
                     r2b2, wout, bout)

    return out[:N, :C]


# speedup vs baseline: 1.0187x; 1.0187x over previous
import functools

import jax
import jax.numpy as jnp
from jax import lax
from jax.experimental import pallas as pl
from jax.experimental.pallas import tpu as pltpu
from jax.experimental.pallas import tpu_sc as plsc

N = 10000
E = 320000
D = 128
C = 47
NPAD = 10240
NC, NS = 2, 16
NW = NC * NS
CHUNK = 125
CPW = 80
EPAD = NW * CHUNK * CPW
GRP = 8
NGRP = CPW // GRP
NBUF = 2
RPS = NPAD // NS
BLK = 512


def _mesh():
    return plsc.VectorSubcoreMesh(core_axis_name="c", subcore_axis_name="s")


def _sc_segment_sum(table, src3, dst3, zrows):
    @functools.partial(
        pl.kernel,
        out_type=jax.ShapeDtypeStruct((NC, NPAD, D), jnp.float32),
        mesh=_mesh(),
        scratch_types=[
            pltpu.VMEM((2, GRP, CHUNK), jnp.int32),
            pltpu.VMEM((2, GRP, CHUNK), jnp.int32),
            pltpu.VMEM((NBUF, CHUNK, D), jnp.float32),
            pltpu.VMEM_SHARED((NPAD, D), jnp.float32),
            [pltpu.SemaphoreType.DMA] * NBUF,
            [pltpu.SemaphoreType.DMA] * 4,
        ],
    )
    def k(table_hbm, src_hbm, dst_hbm, z_hbm, out_hbm, sgrp, dgrp, rows, acc,
          gsems, isems):
        c = lax.axis_index("c")
        s = lax.axis_index("s")
        wid = s * NC + c

        def sload(g, slot):
            return pltpu.make_async_copy(
                src_hbm.at[wid].at[pl.ds(g * GRP, GRP)], sgrp.at[slot],
                isems[slot])

        def dload(g, slot):
            return pltpu.make_async_copy(
                dst_hbm.at[wid].at[pl.ds(g * GRP, GRP)], dgrp.at[slot],
                isems[2 + slot])

        pltpu.sync_copy(z_hbm, acc.at[pl.ds(s * RPS, RPS)])
        sload(0, 0).start()
        dload(0, 0).start()
        plsc.subcore_barrier()

        @pl.loop(0, NGRP // 2)
        def _(q):
            for par in range(2):
                g = 2 * q + par
                sload(g, par).wait()
                dload(g, par).wait()

                @pl.when(g + 1 < NGRP)
                def _():
                    sload(g + 1, 1 - par).start()
                    dload(g + 1, 1 - par).start()

                handles = [
                    pltpu.async_copy(table_hbm.at[sgrp.at[par].at[b]],
                                     rows.at[b], gsems[b])
                    for b in range(NBUF)
                ]
                for k_ in range(GRP):
                    b = k_ % NBUF
                    handles[b].wait()
                    pltpu.sync_copy(rows.at[b], acc.at[dgrp.at[par].at[k_]],
                                    add=True)
                    if k_ + NBUF < GRP:
                        handles[b] = pltpu.async_copy(
                            table_hbm.at[sgrp.at[par].at[k_ + NBUF]],
                            rows.at[b], gsems[b])

        plsc.subcore_barrier()
        pltpu.sync_copy(acc.at[pl.ds(s * RPS, RPS)],
                        out_hbm.at[c].at[pl.ds(s * RPS, RPS)])

    return k(table, src3, dst3, zrows)


def _sc_degree(dst3, ones_rows, zrows):
    @functools.partial(
        pl.kernel,
        out_type=jax.ShapeDtypeStruct((NC, NPAD, D), jnp.float32),
        mesh=_mesh(),
        scratch_types=[
            pltpu.VMEM((CPW, CHUNK), jnp.int32),
            pltpu.VMEM((CHUNK, D), jnp.float32),
            pltpu.VMEM_SHARED((NPAD, D), jnp.float32),
        ],
    )
    def k(dst_hbm, ones_hbm, z_hbm, out_hbm, didx, ones_v, acc):
        c = lax.axis_index("c")
        s = lax.axis_index("s")
        wid = s * NC + c

        pltpu.sync_copy(z_hbm, acc.at[pl.ds(s * RPS, RPS)])
        pltpu.sync_copy(dst_hbm.at[wid], didx)
        pltpu.sync_copy(ones_hbm, ones_v)
        plsc.subcore_barrier()

        @pl.loop(0, CPW)
        def _(j):
            pltpu.sync_copy(ones_v, acc.at[didx.at[j]], add=True)

        plsc.subcore_barrier()
        pltpu.sync_copy(acc.at[pl.ds(s * RPS, RPS)],
                        out_hbm.at[c].at[pl.ds(s * RPS, RPS)])

    return k(dst3, ones_rows, zrows)


def _dot(a, b):
    return jnp.dot(a, b, preferred_element_type=jnp.float32)


def _layer1_body(x_ref, p_ref, dp_ref, ws_ref, wn_ref, b_ref, o_ref):
    agg = p_ref[0] + p_ref[1]
    deg = dp_ref[0][:, :1] + dp_ref[1][:, :1]
    recip = 1.0 / jnp.maximum(deg, 1.0)
    mean = agg * recip
    o_ref[...] = jnp.maximum(_dot(x_ref[...], ws_ref[...])
                             + _dot(mean, wn_ref[...]) + b_ref[...], 0.0)


def _layer2_body(h_ref, p_ref, dp_ref, ws_ref, wn_ref, b_ref, mwin_ref,
                 mbin_ref, r1w1_ref, r1b1_ref, r1w2_ref, r1b2_ref, r2w1_ref,
                 r2b1_ref, r2w2_ref, r2b2_ref, wout_ref, bout_ref, o_ref):
    agg = p_ref[0] + p_ref[1]
    deg = dp_ref[0][:, :1] + dp_ref[1][:, :1]
    recip = 1.0 / jnp.maximum(deg, 1.0)
    mean = agg * recip
    z = _dot(h_ref[...], ws_ref[...]) + _dot(mean, wn_ref[...]) + b_ref[...]
    t = jnp.maximum(_dot(z, mwin_ref[...]) + mbin_ref[...], 0.0)
    t = t + _dot(jnp.maximum(_dot(t, r1w1_ref[...]) + r1b1_ref[...], 0.0),
                 r1w2_ref[...]) + r1b2_ref[...]
    t = t + _dot(jnp.maximum(_dot(t, r2w1_ref[...]) + r2b1_ref[...], 0.0),
                 r2w2_ref[...]) + r2b2_ref[...]
    o_ref[...] = _dot(t, wout_ref[...]) + bout_ref[...]


def _row_spec(width):
    return pl.BlockSpec((BLK, width), lambda i: (i, 0))


def _part_spec():
    return pl.BlockSpec((NC, BLK, D), lambda i: (0, i, 0))


def _w_spec(shape):
    return pl.BlockSpec(shape, lambda i: (0,) * len(shape))


def _tc_layer1(xp, parts, degparts, ws, wn, b):
    return pl.pallas_call(
        _layer1_body,
        grid=(NPAD // BLK,),
        in_specs=[_row_spec(D), _part_spec(), _part_spec(), _w_spec((D, D)),
                  _w_spec((D, D)), _w_spec((1, D))],
        out_specs=_row_spec(D),
        out_shape=jax.ShapeDtypeStruct((NPAD, D), jnp.float32),
    )(xp, parts, degparts, ws, wn, b)


def _tc_layer2(h, parts, degparts, ws, wn, b, mwin, mbin, r1w1, r1b1, r1w2,
               r1b2, r2w1, r2b1, r2w2, r2b2, wout, bout):
    wspecs = [_w_spec((D, D)), _w_spec((D, D)), _w_spec((1, D)),
              _w_spec((D, D)), _w_spec((1, D)),
              _w_spec((D, D)), _w_spec((1, D)), _w_spec((D, D)),
              _w_spec((1, D)),
              _w_spec((D, D)), _w_spec((1, D)), _w_spec((D, D)),
              _w_spec((1, D)),
              _w_spec((D, D)), _w_spec((1, D))]
    return pl.pallas_call(
        _layer2_body,
        grid=(NPAD // BLK,),
        in_specs=[_row_spec(D), _part_spec(), _part_spec()] + wspecs,
        out_specs=_row_spec(D),
        out_shape=jax.ShapeDtypeStruct((NPAD, D), jnp.float32),
    )(h, parts, degparts, ws, wn, b, mwin, mbin, r1w1, r1b1, r1w2, r1b2,
      r2w1, r2b1, r2w2, r2b2, wout, bout)


def kernel(x, edge_index, s1_wself, s1_wneigh, s1_b, s2_wself, s2_wneigh,
           s2_b, m_win, m_bin, r1_w1, r1_b1, r1_w2, r1_b2, r2_w1, r2_b1,
           r2_w2, r2_b2, m_wout, m_bout):
    src = edge_index[0]
    dst = edge_index[1]
    pad_e = EPAD - E
    pad_src = jnp.arange(pad_e, dtype=jnp.int32) % N
    src3 = jnp.concatenate([src, pad_src]).reshape(NW, CPW, CHUNK)
    pad_dst = N + jnp.arange(pad_e, dtype=jnp.int32) % (NPAD - N)
    dst3 = jnp.concatenate([dst, pad_dst]).reshape(NW, CPW, CHUNK)

    xp = jnp.zeros((NPAD, D), jnp.float32).at[:N].set(x)
    zrows = jnp.zeros((RPS, D), jnp.float32)
    ones_rows = jnp.ones((CHUNK, D), jnp.float32)

    b1 = s1_b.reshape(1, D)
    b2 = s2_b.reshape(1, D)
    mbin = m_bin.reshape(1, D)
    r1b1 = r1_b1.reshape(1, D)
    r1b2 = r1_b2.reshape(1, D)
    r2b1 = r2_b1.reshape(1, D)
    r2b2 = r2_b2.reshape(1, D)
    wout = jnp.zeros((D, D), jnp.float32).at[:, :C].set(m_wout)
    bout = jnp.zeros((1, D), jnp.float32).at[0, :C].set(m_bout)

    degparts = _sc_degree(dst3, ones_rows, zrows)
    parts1 = _sc_segment_sum(xp, src3, dst3, zrows)
    h = _tc_layer1(xp, parts1, degparts, s1_wself, s1_wneigh, b1)
    parts2 = _sc_segment_sum(h, src3, dst3, zrows)
    out = _tc_layer2(h, parts2, degparts, s2_wself, s2_wneigh, b2, m_win,
                     mbin, r1_w1, r1b1, r1_w2, r1b2, r2_w1, r2b1, r2_w2,
                     r2b2, wout, bout)

    return out[:N, :C]


# speedup vs baseline: 1.2545x; 1.2316x over previous
import dataclasses
import functools

import jax
import jax.numpy as jnp
from jax import lax
from jax.experimental import pallas as pl
from jax.experimental.pallas import tpu as pltpu
from jax.experimental.pallas import tpu_sc as plsc

N = 10000
E = 320000
D = 128
C = 47
NPAD = 10240
NC, NS = 2, 16
NW = NC * NS
CHUNK = 128
CPW = 80
EPAD = NW * CHUNK * CPW
GRP = 8
NGRP = CPW // GRP
NBUF = 2
RPS = NPAD // NS
BLK = 512


def _mesh():
    return plsc.VectorSubcoreMesh(core_axis_name="c", subcore_axis_name="s")


def _sc_segment_sum(table, src3, dst3, zrows, with_hist):
    out_type = [jax.ShapeDtypeStruct((NC, NPAD, D), jnp.float32)]
    scratch = [
        pltpu.VMEM((2, GRP, CHUNK), jnp.int32),
        pltpu.VMEM((2, GRP, CHUNK), jnp.int32),
        pltpu.VMEM((NBUF, CHUNK, D), jnp.float32),
        pltpu.VMEM_SHARED((NPAD, D), jnp.float32),
        [pltpu.SemaphoreType.DMA] * NBUF,
        [pltpu.SemaphoreType.DMA] * 4,
    ]
    if with_hist:
        out_type.append(jax.ShapeDtypeStruct((NC, NS, NPAD), jnp.float32))
        scratch.append(pltpu.VMEM((NPAD,), jnp.float32))

    cp = pltpu.CompilerParams()
    if "needs_layout_passes" in pltpu.CompilerParams.__dataclass_fields__:
        cp = dataclasses.replace(cp, needs_layout_passes=False)

    @functools.partial(pl.kernel,
                       out_type=tuple(out_type) if with_hist else out_type[0],
                       mesh=_mesh(), scratch_types=scratch,
                       compiler_params=cp)
    def k(table_hbm, src_hbm, dst_hbm, z_hbm, *rest):
        if with_hist:
            out_hbm, hout_hbm, sgrp, dgrp, rows, acc, gsems, isems, hist = rest
        else:
            out_hbm, sgrp, dgrp, rows, acc, gsems, isems = rest
            hist = None
        c = lax.axis_index("c")
        s = lax.axis_index("s")
        wid = s * NC + c

        def sload(g, slot):
            return pltpu.make_async_copy(
                src_hbm.at[wid].at[pl.ds(g * GRP, GRP)], sgrp.at[slot],
                isems[slot])

        def dload(g, slot):
            return pltpu.make_async_copy(
                dst_hbm.at[wid].at[pl.ds(g * GRP, GRP)], dgrp.at[slot],
                isems[2 + slot])

        pltpu.sync_copy(z_hbm, acc.at[pl.ds(s * RPS, RPS)])
        sload(0, 0).start()
        dload(0, 0).start()
        if with_hist:
            zero16 = jnp.zeros((16,), jnp.float32)

            @pl.loop(0, NPAD // 16)
            def _(i):
                hist[pl.ds(i * 16, 16)] = zero16

        plsc.subcore_barrier()

        @pl.loop(0, NGRP // 2)
        def _(q):
            for par in range(2):
                g = 2 * q + par
                sload(g, par).wait()
                dload(g, par).wait()

                @pl.when(g + 1 < NGRP)
                def _():
                    sload(g + 1, 1 - par).start()
                    dload(g + 1, 1 - par).start()

                handles = [
                    pltpu.async_copy(table_hbm.at[sgrp.at[par].at[b]],
                                     rows.at[b], gsems[b])
                    for b in range(NBUF)
                ]
                for k_ in range(GRP):
                    b = k_ % NBUF
                    handles[b].wait()
                    pltpu.sync_copy(rows.at[b], acc.at[dgrp.at[par].at[k_]],
                                    add=True)
                    if k_ + NBUF < GRP:
                        handles[b] = pltpu.async_copy(
                            table_hbm.at[sgrp.at[par].at[k_ + NBUF]],
                            rows.at[b], gsems[b])
                    if with_hist:
                        one16 = jnp.ones((16,), jnp.float32)
                        for v in range(CHUNK // 16):
                            idx16 = dgrp[par, k_, pl.ds(v * 16, 16)]
                            plsc.addupdate_scatter(hist, [idx16], one16)

        plsc.subcore_barrier()
        pltpu.sync_copy(acc.at[pl.ds(s * RPS, RPS)],
                        out_hbm.at[c].at[pl.ds(s * RPS, RPS)])
        if with_hist:
            pltpu.sync_copy(hist, hout_hbm.at[c].at[s])

    return k(table, src3, dst3, zrows)


def _dot(a, b):
    return jnp.dot(a, b, preferred_element_type=jnp.float32)


def _layer1_body(x_ref, p_ref, dp_ref, ws_ref, wn_ref, b_ref, o_ref):
    agg = p_ref[0] + p_ref[1]
    deg = jnp.sum(dp_ref[...], axis=(0, 1))[:, None]
    recip = 1.0 / jnp.maximum(deg, 1.0)
    mean = agg * recip
    o_ref[...] = jnp.maximum(_dot(x_ref[...], ws_ref[...])
                             + _dot(mean, wn_ref[...]) + b_ref[...], 0.0)


def _layer2_body(h_ref, p_ref, dp_ref, ws_ref, wn_ref, b_ref, mwin_ref,
                 mbin_ref, r1w1_ref, r1b1_ref, r1w2_ref, r1b2_ref, r2w1_ref,
                 r2b1_ref, r2w2_ref, r2b2_ref, wout_ref, bout_ref, o_ref):
    agg = p_ref[0] + p_ref[1]
    deg = jnp.sum(dp_ref[...], axis=(0, 1))[:, None]
    recip = 1.0 / jnp.maximum(deg, 1.0)
    mean = agg * recip
    z = _dot(h_ref[...], ws_ref[...]) + _dot(mean, wn_ref[...]) + b_ref[...]
    t = jnp.maximum(_dot(z, mwin_ref[...]) + mbin_ref[...], 0.0)
    t = t + _dot(jnp.maximum(_dot(t, r1w1_ref[...]) + r1b1_ref[...], 0.0),
                 r1w2_ref[...]) + r1b2_ref[...]
    t = t + _dot(jnp.maximum(_dot(t, r2w1_ref[...]) + r2b1_ref[...], 0.0),
                 r2w2_ref[...]) + r2b2_ref[...]
    o_ref[...] = _dot(t, wout_ref[...]) + bout_ref[...]


def _row_spec(width):
    return pl.BlockSpec((BLK, width), lambda i: (i, 0))


def _part_spec():
    return pl.BlockSpec((NC, BLK, D), lambda i: (0, i, 0))


def _w_spec(shape):
    return pl.BlockSpec(shape, lambda i: (0,) * len(shape))


def _hist_spec():
    return pl.BlockSpec((NC, NS, BLK), lambda i: (0, 0, i))


def _tc_layer1(xp, parts, hists, ws, wn, b):
    return pl.pallas_call(
        _layer1_body,
        grid=(NPAD // BLK,),
        in_specs=[_row_spec(D), _part_spec(), _hist_spec(), _w_spec((D, D)),
                  _w_spec((D, D)), _w_spec((1, D))],
        out_specs=_row_spec(D),
        out_shape=jax.ShapeDtypeStruct((NPAD, D), jnp.float32),
    )(xp, parts, hists, ws, wn, b)


def _tc_layer2(h, parts, hists, ws, wn, b, mwin, mbin, r1w1, r1b1, r1w2,
               r1b2, r2w1, r2b1, r2w2, r2b2, wout, bout):
    wspecs = [_w_spec((D, D)), _w_spec((D, D)), _w_spec((1, D)),
              _w_spec((D, D)), _w_spec((1, D)),
              _w_spec((D, D)), _w_spec((1, D)), _w_spec((D, D)),
              _w_spec((1, D)),
              _w_spec((D, D)), _w_spec((1, D)), _w_spec((D, D)),
              _w_spec((1, D)),
              _w_spec((D, D)), _w_spec((1, D))]
    return pl.pallas_call(
        _layer2_body,
        grid=(NPAD // BLK,),
        in_specs=[_row_spec(D), _part_spec(), _hist_spec()] + wspecs,
        out_specs=_row_spec(D),
        out_shape=jax.ShapeDtypeStruct((NPAD, D), jnp.float32),
    )(h, parts, hists, ws, wn, b, mwin, mbin, r1w1, r1b1, r1w2, r1b2,
      r2w1, r2b1, r2w2, r2b2, wout, bout)


def kernel(x, edge_index, s1_wself, s1_wneigh, s1_b, s2_wself, s2_wneigh,
           s2_b, m_win, m_bin, r1_w1, r1_b1, r1_w2, r1_b2, r2_w1, r2_b1,
           r2_w2, r2_b2, m_wout, m_bout):
    src = edge_index[0]
    dst = edge_index[1]
    pad_e = EPAD - E
    pad_src = jnp.arange(pad_e, dtype=jnp.int32) % N
    src3 = jnp.concatenate([src, pad_src]).reshape(NW, CPW, CHUNK)
    pad_dst = N + jnp.arange(pad_e, dtype=jnp.int32) % (NPAD - N)
    dst3 = jnp.concatenate([dst, pad_dst]).reshape(NW, CPW, CHUNK)

    xp = jnp.zeros((NPAD, D), jnp.float32).at[:N].set(x)
    zrows = jnp.zeros((RPS, D), jnp.float32)

    b1 = s1_b.reshape(1, D)
    b2 = s2_b.reshape(1, D)
    mbin = m_bin.reshape(1, D)
    r1b1 = r1_b1.reshape(1, D)
    r1b2 = r1_b2.reshape(1, D)
    r2b1 = r2_b1.reshape(1, D)
    r2b2 = r2_b2.reshape(1, D)
    wout = jnp.zeros((D, D), jnp.float32).at[:, :C].set(m_wout)
    bout = jnp.zeros((1, D), jnp.float32).at[0, :C].set(m_bout)

    parts1, hists = _sc_segment_sum(xp, src3, dst3, zrows, True)
    h = _tc_layer1(xp, parts1, hists, s1_wself, s1_wneigh, b1)
    parts2 = _sc_segment_sum(h, src3, dst3, zrows, False)
    out = _tc_layer2(h, parts2, hists, s2_wself, s2_wneigh, b2, m_win,
                     mbin, r1_w1, r1b1, r1_w2, r1b2, r2_w1, r2b1, r2_w2,
                     r2b2, wout, bout)

    return out[:N, :C]
